# baseline (speedup 1.0000x reference)
import functools

import jax
import jax.numpy as jnp
from jax import lax
from jax.experimental import pallas as pl
from jax.experimental.pallas import tpu as pltpu
from jax.experimental.pallas import tpu_sc as plsc

_NC = 2
_NS = 16
_NW = _NC * _NS
_NBUF = 12
_LANES = 16


def _make_lookup(B, V, D):
    b_per_w = B // _NW
    half = b_per_w // 2

    mesh = plsc.VectorSubcoreMesh(core_axis_name="c", subcore_axis_name="s")

    @functools.partial(
        pl.kernel,
        mesh=mesh,
        out_type=jax.ShapeDtypeStruct((D, B), jnp.float32),
        scratch_types=[
            pltpu.VMEM((b_per_w + _LANES,), jnp.int32),
            pltpu.VMEM((_NBUF, D, 128), jnp.float32),
            pltpu.VMEM((D, half), jnp.float32),
        ]
        + [pltpu.SemaphoreType.DMA] * _NBUF,
        compiler_params=pltpu.CompilerParams(
            needs_layout_passes=False, internal_scratch_in_bytes=524288
        ),
    )
    def emb_lookup(idx_hbm, tableT_hbm, outT_hbm, idx_v, blocks_v, cols_v, *sems):
        wid = lax.axis_index("s") * _NC + lax.axis_index("c")
        base = wid * b_per_w
        pltpu.sync_copy(
            idx_hbm.at[pl.ds(base, b_per_w)], idx_v.at[pl.ds(0, b_per_w)]
        )

        def fetch(i, b):
            s = idx_v[pl.ds(i, _LANES)][0]
            cb = pl.multiple_of(s - lax.rem(s, 128), 128)
            pltpu.async_copy(
                tableT_hbm.at[:, pl.ds(cb, 128)], blocks_v.at[b], sems[b]
            )

        for b in range(_NBUF):
            fetch(b, b)

        def consume(i, b):
            @pl.when(i == half)
            def _():
                pltpu.sync_copy(cols_v, outT_hbm.at[:, pl.ds(base, half)])

            pltpu.make_async_copy(
                tableT_hbm.at[:, pl.ds(0, 128)], blocks_v.at[b], sems[b]
            ).wait()
            s = idx_v[pl.ds(i, _LANES)][0]
            lane = lax.rem(s, 128)
            col = lax.broadcast(lane, (_LANES,))
            icol = lax.broadcast(lax.rem(i, half), (_LANES,))
            for c in range(D // _LANES):
                row = lax.iota(jnp.int32, _LANES) + c * _LANES
                vec = plsc.load_gather(blocks_v.at[b], [row, col])
                plsc.store_scatter(cols_v, [row, icol], vec)

            @pl.when(i + _NBUF < b_per_w)
            def _():
                fetch(i + _NBUF, b)

        n_groups = b_per_w // _NBUF
        tail = b_per_w - n_groups * _NBUF

        def group(g, carry):
            for b in range(_NBUF):
                consume(g * _NBUF + b, b)
            return carry

        lax.fori_loop(0, n_groups, group, 0)
        for t in range(tail):
            consume(n_groups * _NBUF + t, (n_groups * _NBUF + t) % _NBUF)
        pltpu.sync_copy(cols_v, outT_hbm.at[:, pl.ds(base + half, half)])

    return emb_lookup


def kernel(inp, table):
    (B,) = inp.shape
    V, D = table.shape
    outT = _make_lookup(B, V, D)(inp.astype(jnp.int32), table.T)
    return outT.T


# speedup vs baseline: 1.0075x; 1.0075x over previous
import functools

import jax
import jax.numpy as jnp
from jax import lax
from jax.experimental import pallas as pl
from jax.experimental.pallas import tpu as pltpu
from jax.experimental.pallas import tpu_sc as plsc

_NC = 2
_NS = 16
_NW = _NC * _NS
_NBUF = 8
_LANES = 16


def _make_lookup(B, V, D):
    b_per_w = B // _NW
    half = b_per_w // 2

    mesh = plsc.VectorSubcoreMesh(core_axis_name="c", subcore_axis_name="s")

    @functools.partial(
        pl.kernel,
        mesh=mesh,
        out_type=jax.ShapeDtypeStruct((D, B), jnp.float32),
        scratch_types=[
            pltpu.VMEM((b_per_w + _LANES,), jnp.int32),
            pltpu.VMEM((_NBUF, D, 128), jnp.float32),
            pltpu.VMEM((D, half), jnp.float32),
        ]
        + [pltpu.SemaphoreType.DMA] * _NBUF,
        compiler_params=pltpu.CompilerParams(needs_layout_passes=False),
    )
    def emb_lookup(idx_hbm, tableT_hbm, outT_hbm, idx_v, blocks_v, cols_v, *sems):
        wid = lax.axis_index("s") * _NC + lax.axis_index("c")
        base = wid * b_per_w
        pltpu.sync_copy(
            idx_hbm.at[pl.ds(base, b_per_w)], idx_v.at[pl.ds(0, b_per_w)]
        )

        def fetch(i, b):
            s = idx_v[pl.ds(i, _LANES)][0]
            cb = pl.multiple_of(s - lax.rem(s, 128), 128)
            pltpu.async_copy(
                tableT_hbm.at[:, pl.ds(cb, 128)], blocks_v.at[b], sems[b]
            )

        for b in range(_NBUF):
            fetch(b, b)

        def consume(i, b):
            @pl.when(i == half)
            def _():
                pltpu.sync_copy(cols_v, outT_hbm.at[:, pl.ds(base, half)])

            pltpu.make_async_copy(
                tableT_hbm.at[:, pl.ds(0, 128)], blocks_v.at[b], sems[b]
            ).wait()
            s = idx_v[pl.ds(i, _LANES)][0]
            lane = lax.rem(s, 128)
            col = lax.broadcast(lane, (_LANES,))
            icol = lax.broadcast(lax.rem(i, half), (_LANES,))
            for c in range(D // _LANES):
                row = lax.iota(jnp.int32, _LANES) + c * _LANES
                vec = plsc.load_gather(blocks_v.at[b], [row, col])
                plsc.store_scatter(cols_v, [row, icol], vec)

            @pl.when(i + _NBUF < b_per_w)
            def _():
                fetch(i + _NBUF, b)

        n_groups = b_per_w // _NBUF
        tail = b_per_w - n_groups * _NBUF

        def group(g, carry):
            for b in range(_NBUF):
                consume(g * _NBUF + b, b)
            return carry

        lax.fori_loop(0, n_groups, group, 0)
        for t in range(tail):
            consume(n_groups * _NBUF + t, (n_groups * _NBUF + t) % _NBUF)
        pltpu.sync_copy(cols_v, outT_hbm.at[:, pl.ds(base + half, half)])

    return emb_lookup


def kernel(inp, table):
    (B,) = inp.shape
    V, D = table.shape
    outT = _make_lookup(B, V, D)(inp.astype(jnp.int32), table.T)
    return outT.T
